# baseline (speedup 1.0000x reference)
import functools

import jax
import jax.numpy as jnp
from jax import lax
from jax.experimental import pallas as pl
from jax.experimental.pallas import tpu as pltpu
from jax.experimental.pallas import tpu_sc as plsc

N = 10000
E_TOT = 320000
D = 128
NC, NS = 2, 16
NW = NC * NS
K = 48
CH = 216
EDGES_PER_W = K * CH
E_PAD = NW * EDGES_PER_W
DUMMY = N
ACC_ROWS = 10240
ROWS_PER_S = ACC_ROWS // NS


def _sc_edge_body(p_hbm, q_hbm, src_hbm, dst_hbm, part_out, cnt_out,
                  srcr, dstr, q0, q1, q2, p0, p1, p2, ones_v, zcnt_v,
                  acc_sh, cnt_sh,
                  si0, si1, si2, sq0, sq1, sq2, sg0, sg1, sg2,
                  ss0, ss1, ss2, sn0, sn1, sn2):
    c = lax.axis_index("c")
    s = lax.axis_index("s")
    w = c * NS + s
    zero16 = jnp.zeros((16,), jnp.float32)
    one16 = jnp.ones((16,), jnp.float32)
    qb = (q0, q1, q2)
    pb = (p0, p1, p2)
    si = (si0, si1, si2)
    sq = (sq0, sq1, sq2)
    sg = (sg0, sg1, sg2)
    ss = (ss0, ss1, ss2)
    sn = (sn0, sn1, sn2)

    def _ones_fill(i, carry):
        ones_v[pl.ds(i * 16, 16)] = one16
        return carry
    lax.fori_loop(0, K // 16, _ones_fill, 0)

    def _zcnt_fill(i, carry):
        zcnt_v[pl.ds(i * 16, 16)] = zero16
        return carry
    lax.fori_loop(0, ROWS_PER_S // 16, _zcnt_fill, 0)

    def _zrow_fill(i, carry):
        for j in range(D // 16):
            q0[i, pl.ds(j * 16, 16)] = zero16
        return carry
    lax.fori_loop(0, K, _zrow_fill, 0)
    for i in range(ROWS_PER_S // K):
        pltpu.sync_copy(q0, acc_sh.at[pl.ds(s * ROWS_PER_S + i * K, K)])
    tail = ROWS_PER_S - (ROWS_PER_S // K) * K
    if tail:
        pltpu.sync_copy(
            q0.at[pl.ds(0, tail)],
            acc_sh.at[pl.ds(s * ROWS_PER_S + (ROWS_PER_S // K) * K, tail)])
    pltpu.sync_copy(zcnt_v, cnt_sh.at[pl.ds(s * ROWS_PER_S, ROWS_PER_S)])
    plsc.subcore_barrier()

    qbase = w * EDGES_PER_W

    def _l1(k, b):
        pltpu.make_async_copy(src_hbm.at[w, k], srcr.at[b], si[b]).start()
        pltpu.make_async_copy(dst_hbm.at[w, k], dstr.at[b], si[b]).start()
        pltpu.make_async_copy(
            q_hbm.at[pl.ds(qbase + k * K, K)], qb[b], sq[b]).start()

    def _wait_idx(k, b):
        pltpu.make_async_copy(src_hbm.at[w, k], srcr.at[b], si[b]).wait()
        pltpu.make_async_copy(dst_hbm.at[w, k], dstr.at[b], si[b]).wait()

    def _g(b):
        pltpu.make_async_copy(p_hbm.at[srcr.at[b]], pb[b], sg[b]).start()

    def _wait_qg(k, b):
        pltpu.make_async_copy(
            q_hbm.at[pl.ds(qbase + k * K, K)], qb[b], sq[b]).wait()
        pltpu.make_async_copy(p_hbm.at[srcr.at[b]], pb[b], sg[b]).wait()

    def _compute(b):
        def _edge(e, cin):
            for j in range(D // 16):
                sl = pl.ds(j * 16, 16)
                qb[b][e, sl] = jnp.maximum(qb[b][e, sl] + pb[b][e, sl], zero16)
            return cin
        lax.fori_loop(0, K, _edge, 0)

    def _scat(b):
        pltpu.make_async_copy(qb[b], acc_sh.at[dstr.at[b]], ss[b]).start(
            add=True)
        pltpu.make_async_copy(ones_v, cnt_sh.at[dstr.at[b]], sn[b]).start(
            add=True)

    def _wait_scat(b):
        pltpu.make_async_copy(qb[b], acc_sh.at[dstr.at[b]], ss[b]).wait()
        pltpu.make_async_copy(ones_v, cnt_sh.at[dstr.at[b]], sn[b]).wait()

    def _body(k, b, first, g_next, load_next):
        b1, b2 = (b + 1) % 3, (b + 2) % 3
        if g_next:
            _wait_idx(k + 1, b1)
            _g(b1)
        _wait_qg(k, b)
        _compute(b)
        _scat(b)
        if not first:
            _wait_scat(b2)
        if load_next:
            _l1(k + 2, b2)

    _l1(0, 0)
    _l1(1, 1)
    _wait_idx(0, 0)
    _g(0)
    _body(0, 0, True, True, True)
    _body(1, 1, False, True, True)
    _body(2, 2, False, True, True)

    def _steady(g, carry):
        k = 3 * g
        _body(k, 0, False, True, True)
        _body(k + 1, 1, False, True, True)
        _body(k + 2, 2, False, True, True)
        return carry
    lax.fori_loop(1, CH // 3 - 1, _steady, 0)

    _body(CH - 3, 0, False, True, True)
    _body(CH - 2, 1, False, True, False)
    _body(CH - 1, 2, False, False, False)
    _wait_scat(2)
    plsc.subcore_barrier()

    pltpu.sync_copy(acc_sh.at[pl.ds(s * ROWS_PER_S, ROWS_PER_S)],
                    part_out.at[c, pl.ds(s * ROWS_PER_S, ROWS_PER_S)])
    pltpu.sync_copy(cnt_sh.at[pl.ds(s * ROWS_PER_S, ROWS_PER_S)],
                    cnt_out.at[c, pl.ds(s * ROWS_PER_S, ROWS_PER_S)])


_sc_edge_pass = functools.partial(
    pl.kernel,
    mesh=plsc.VectorSubcoreMesh(core_axis_name="c", subcore_axis_name="s"),
    out_type=[jax.ShapeDtypeStruct((NC, ACC_ROWS, D), jnp.float32),
              jax.ShapeDtypeStruct((NC, ACC_ROWS), jnp.float32)],
    scratch_types=[
        pltpu.VMEM((3, K), jnp.int32),
        pltpu.VMEM((3, K), jnp.int32),
        pltpu.VMEM((K, D), jnp.float32),
        pltpu.VMEM((K, D), jnp.float32),
        pltpu.VMEM((K, D), jnp.float32),
        pltpu.VMEM((K, D), jnp.float32),
        pltpu.VMEM((K, D), jnp.float32),
        pltpu.VMEM((K, D), jnp.float32),
        pltpu.VMEM((K,), jnp.float32),
        pltpu.VMEM((ROWS_PER_S,), jnp.float32),
        pltpu.VMEM_SHARED((ACC_ROWS, D), jnp.float32),
        pltpu.VMEM_SHARED((ACC_ROWS,), jnp.float32),
    ] + [pltpu.SemaphoreType.DMA] * 15,
)(_sc_edge_body)


def _mm_block(x_ref, w_ref, b_ref, o_ref):
    o_ref[...] = jnp.dot(x_ref[...], w_ref[...],
                         preferred_element_type=jnp.float32) + b_ref[...]


def _matmul(x, w, b, bm):
    m, kdim = x.shape
    n = w.shape[1]
    return pl.pallas_call(
        _mm_block,
        grid=(m // bm,),
        in_specs=[pl.BlockSpec((bm, kdim), lambda i: (i, 0)),
                  pl.BlockSpec((kdim, n), lambda i: (0, 0)),
                  pl.BlockSpec((1, n), lambda i: (0, 0))],
        out_specs=pl.BlockSpec((bm, n), lambda i: (i, 0)),
        out_shape=jax.ShapeDtypeStruct((m, n), jnp.float32),
    )(x, w, b.reshape(1, n))


def _combine_block(r_ref, p0_ref, p1_ref, c0_ref, c1_ref, o_ref):
    cnt = jnp.maximum(c0_ref[...] + c1_ref[...], 1.0)
    agg = (p0_ref[0] + p1_ref[0]) / cnt
    o_ref[...] = jnp.maximum(r_ref[...] + agg, 0.0)


def _combine(r, parts, cnts, bm=1000):
    c0 = cnts[0].reshape(ACC_ROWS, 1)
    c1 = cnts[1].reshape(ACC_ROWS, 1)
    return pl.pallas_call(
        _combine_block,
        grid=(N // bm,),
        in_specs=[pl.BlockSpec((bm, D), lambda i: (i, 0)),
                  pl.BlockSpec((1, bm, D), lambda i: (0, i, 0)),
                  pl.BlockSpec((1, bm, D), lambda i: (1, i, 0)),
                  pl.BlockSpec((bm, 1), lambda i: (i, 0)),
                  pl.BlockSpec((bm, 1), lambda i: (i, 0))],
        out_specs=pl.BlockSpec((bm, D), lambda i: (i, 0)),
        out_shape=jax.ShapeDtypeStruct((N, D), jnp.float32),
    )(r, parts, parts, c0, c1)


def kernel(x_vals, x_cons, edge_attr_v2c, edge_attr_c2v,
           edge_index_v2c, edge_index_c2v, batch_vals, batch_cons,
           W_msg_v2c, W_root_v2c, b_v2c, W_msg_c2v, W_root_c2v, b_c2v):
    del batch_vals, batch_cons
    zb = jnp.zeros((D,), jnp.float32)
    e1 = jnp.pad(edge_attr_v2c, ((0, E_PAD - E_TOT), (0, 4)))
    e2 = jnp.pad(edge_attr_c2v, ((0, E_PAD - E_TOT), (0, 4)))
    B1 = jnp.pad(W_msg_v2c[D:], ((0, 4), (0, 0)))
    B2 = jnp.pad(W_msg_c2v[D:], ((0, 4), (0, 0)))

    P1 = _matmul(x_vals, W_msg_v2c[:D], zb, 1000)
    Q1 = _matmul(e1, B1, zb, 2048)
    R1 = _matmul(x_cons, W_root_v2c, b_v2c, 1000)
    Q2 = _matmul(e2, B2, zb, 2048)
    R2 = _matmul(x_vals, W_root_c2v, b_c2v, 1000)

    def _prep_idx(row, fill):
        p = jnp.pad(row.astype(jnp.int32), (0, E_PAD - E_TOT),
                    constant_values=fill)
        return p.reshape(NW, CH, K)

    src1 = _prep_idx(edge_index_v2c[0], 0)
    dst1 = _prep_idx(edge_index_v2c[1], DUMMY)
    src2 = _prep_idx(edge_index_c2v[0], 0)
    dst2 = _prep_idx(edge_index_c2v[1], DUMMY)

    part1, cnt1 = _sc_edge_pass(P1, Q1, src1, dst1)
    x_cons_new = _combine(R1, part1, cnt1)

    P2 = _matmul(x_cons_new, W_msg_c2v[:D], zb, 1000)
    part2, cnt2 = _sc_edge_pass(P2, Q2, src2, dst2)
    x_vals_new = _combine(R2, part2, cnt2)

    return (x_vals_new, x_cons_new)


# speedup vs baseline: 1.0005x; 1.0005x over previous
import functools

import jax
import jax.numpy as jnp
from jax import lax
from jax.experimental import pallas as pl
from jax.experimental.pallas import tpu as pltpu
from jax.experimental.pallas import tpu_sc as plsc

N = 10000
E_TOT = 320000
D = 128
NC, NS = 2, 16
NW = NC * NS
K = 48
CH = 216
EDGES_PER_W = K * CH
E_PAD = NW * EDGES_PER_W
DUMMY = N
ACC_ROWS = 10240
ROWS_PER_S = ACC_ROWS // NS


def _sc_edge_body(p_hbm, q_hbm, src_hbm, dst_hbm, part_out, cnt_out,
                  srcr, dstr, q0, q1, q2, p0, p1, p2, ones_v, zcnt_v,
                  acc_sh, cnt_sh,
                  si0, si1, si2, sq0, sq1, sq2, sg0, sg1, sg2,
                  ss0, ss1, ss2, sn0, sn1, sn2):
    c = lax.axis_index("c")
    s = lax.axis_index("s")
    w = c * NS + s
    zero16 = jnp.zeros((16,), jnp.float32)
    one16 = jnp.ones((16,), jnp.float32)
    qb = (q0, q1, q2)
    pb = (p0, p1, p2)
    si = (si0, si1, si2)
    sq = (sq0, sq1, sq2)
    sg = (sg0, sg1, sg2)
    ss = (ss0, ss1, ss2)
    sn = (sn0, sn1, sn2)

    def _ones_fill(i, carry):
        ones_v[pl.ds(i * 16, 16)] = one16
        return carry
    lax.fori_loop(0, K // 16, _ones_fill, 0)

    def _zcnt_fill(i, carry):
        zcnt_v[pl.ds(i * 16, 16)] = zero16
        return carry
    lax.fori_loop(0, ROWS_PER_S // 16, _zcnt_fill, 0)

    def _zrow_fill(i, carry):
        for j in range(D // 16):
            q0[i, pl.ds(j * 16, 16)] = zero16
        return carry
    lax.fori_loop(0, K, _zrow_fill, 0)
    for i in range(ROWS_PER_S // K):
        pltpu.sync_copy(q0, acc_sh.at[pl.ds(s * ROWS_PER_S + i * K, K)])
    tail = ROWS_PER_S - (ROWS_PER_S // K) * K
    if tail:
        pltpu.sync_copy(
            q0.at[pl.ds(0, tail)],
            acc_sh.at[pl.ds(s * ROWS_PER_S + (ROWS_PER_S // K) * K, tail)])
    pltpu.sync_copy(zcnt_v, cnt_sh.at[pl.ds(s * ROWS_PER_S, ROWS_PER_S)])
    plsc.subcore_barrier()

    qbase = w * EDGES_PER_W

    def _l1(k, b):
        pltpu.make_async_copy(src_hbm.at[w, k], srcr.at[b], si[b]).start()
        pltpu.make_async_copy(dst_hbm.at[w, k], dstr.at[b], si[b]).start()
        pltpu.make_async_copy(
            q_hbm.at[pl.ds(qbase + k * K, K)], qb[b], sq[b]).start()

    def _wait_idx(k, b):
        pltpu.make_async_copy(src_hbm.at[w, k], srcr.at[b], si[b]).wait()
        pltpu.make_async_copy(dst_hbm.at[w, k], dstr.at[b], si[b]).wait()

    def _g(b):
        pltpu.make_async_copy(p_hbm.at[srcr.at[b]], pb[b], sg[b]).start()

    def _wait_qg(k, b):
        pltpu.make_async_copy(
            q_hbm.at[pl.ds(qbase + k * K, K)], qb[b], sq[b]).wait()
        pltpu.make_async_copy(p_hbm.at[srcr.at[b]], pb[b], sg[b]).wait()

    def _compute(b):
        def _edge(e, cin):
            for j in range(D // 16):
                sl = pl.ds(j * 16, 16)
                qb[b][e, sl] = jnp.maximum(qb[b][e, sl] + pb[b][e, sl], zero16)
            return cin
        lax.fori_loop(0, K, _edge, 0)

    def _scat(b):
        pltpu.make_async_copy(qb[b], acc_sh.at[dstr.at[b]], ss[b]).start(
            add=True)
        pltpu.make_async_copy(ones_v, cnt_sh.at[dstr.at[b]], sn[b]).start(
            add=True)

    def _wait_scat(b):
        pltpu.make_async_copy(qb[b], acc_sh.at[dstr.at[b]], ss[b]).wait()
        pltpu.make_async_copy(ones_v, cnt_sh.at[dstr.at[b]], sn[b]).wait()

    def _body(k, b, first, g_next, load_next):
        b1, b2 = (b + 1) % 3, (b + 2) % 3
        if g_next:
            _wait_idx(k + 1, b1)
            _g(b1)
        _wait_qg(k, b)
        _compute(b)
        _scat(b)
        if not first:
            _wait_scat(b2)
        if load_next:
            _l1(k + 2, b2)

    _l1(0, 0)
    _l1(1, 1)
    _wait_idx(0, 0)
    _g(0)
    _body(0, 0, True, True, True)
    _body(1, 1, False, True, True)
    _body(2, 2, False, True, True)

    def _steady(g, carry):
        k = 3 * g
        _body(k, 0, False, True, True)
        _body(k + 1, 1, False, True, True)
        _body(k + 2, 2, False, True, True)
        return carry
    lax.fori_loop(1, CH // 3 - 1, _steady, 0)

    _body(CH - 3, 0, False, True, True)
    _body(CH - 2, 1, False, True, False)
    _body(CH - 1, 2, False, False, False)
    _wait_scat(2)
    plsc.subcore_barrier()

    pltpu.sync_copy(acc_sh.at[pl.ds(s * ROWS_PER_S, ROWS_PER_S)],
                    part_out.at[c, pl.ds(s * ROWS_PER_S, ROWS_PER_S)])
    pltpu.sync_copy(cnt_sh.at[pl.ds(s * ROWS_PER_S, ROWS_PER_S)],
                    cnt_out.at[c, pl.ds(s * ROWS_PER_S, ROWS_PER_S)])


_sc_edge_pass = functools.partial(
    pl.kernel,
    mesh=plsc.VectorSubcoreMesh(core_axis_name="c", subcore_axis_name="s"),
    out_type=[jax.ShapeDtypeStruct((NC, ACC_ROWS, D), jnp.float32),
              jax.ShapeDtypeStruct((NC, ACC_ROWS), jnp.float32)],
    scratch_types=[
        pltpu.VMEM((3, K), jnp.int32),
        pltpu.VMEM((3, K), jnp.int32),
        pltpu.VMEM((K, D), jnp.float32),
        pltpu.VMEM((K, D), jnp.float32),
        pltpu.VMEM((K, D), jnp.float32),
        pltpu.VMEM((K, D), jnp.float32),
        pltpu.VMEM((K, D), jnp.float32),
        pltpu.VMEM((K, D), jnp.float32),
        pltpu.VMEM((K,), jnp.float32),
        pltpu.VMEM((ROWS_PER_S,), jnp.float32),
        pltpu.VMEM_SHARED((ACC_ROWS, D), jnp.float32),
        pltpu.VMEM_SHARED((ACC_ROWS,), jnp.float32),
    ] + [pltpu.SemaphoreType.DMA] * 15,
)(_sc_edge_body)


def _mm_block(x_ref, w_ref, b_ref, o_ref):
    o_ref[...] = jnp.dot(x_ref[...], w_ref[...],
                         preferred_element_type=jnp.float32) + b_ref[...]


def _matmul(x, w, b, bm):
    m, kdim = x.shape
    n = w.shape[1]
    return pl.pallas_call(
        _mm_block,
        grid=(m // bm,),
        in_specs=[pl.BlockSpec((bm, kdim), lambda i: (i, 0)),
                  pl.BlockSpec((kdim, n), lambda i: (0, 0)),
                  pl.BlockSpec((1, n), lambda i: (0, 0))],
        out_specs=pl.BlockSpec((bm, n), lambda i: (i, 0)),
        out_shape=jax.ShapeDtypeStruct((m, n), jnp.float32),
    )(x, w, b.reshape(1, n))


def _combine_block(r_ref, p0_ref, p1_ref, c0_ref, c1_ref, o_ref):
    cnt = jnp.maximum(c0_ref[...] + c1_ref[...], 1.0)
    agg = (p0_ref[0] + p1_ref[0]) / cnt
    o_ref[...] = jnp.maximum(r_ref[...] + agg, 0.0)


def _combine(r, parts, cnts, bm=1000):
    c0 = cnts[0].reshape(ACC_ROWS, 1)
    c1 = cnts[1].reshape(ACC_ROWS, 1)
    return pl.pallas_call(
        _combine_block,
        grid=(N // bm,),
        in_specs=[pl.BlockSpec((bm, D), lambda i: (i, 0)),
                  pl.BlockSpec((1, bm, D), lambda i: (0, i, 0)),
                  pl.BlockSpec((1, bm, D), lambda i: (1, i, 0)),
                  pl.BlockSpec((bm, 1), lambda i: (i, 0)),
                  pl.BlockSpec((bm, 1), lambda i: (i, 0))],
        out_specs=pl.BlockSpec((bm, D), lambda i: (i, 0)),
        out_shape=jax.ShapeDtypeStruct((N, D), jnp.float32),
    )(r, parts, parts, c0, c1)


def kernel(x_vals, x_cons, edge_attr_v2c, edge_attr_c2v,
           edge_index_v2c, edge_index_c2v, batch_vals, batch_cons,
           W_msg_v2c, W_root_v2c, b_v2c, W_msg_c2v, W_root_c2v, b_c2v):
    del batch_vals, batch_cons
    zb = jnp.zeros((D,), jnp.float32)
    e1 = jnp.pad(edge_attr_v2c, ((0, E_PAD - E_TOT), (0, 4)))
    e2 = jnp.pad(edge_attr_c2v, ((0, E_PAD - E_TOT), (0, 4)))
    B1 = jnp.pad(W_msg_v2c[D:], ((0, 4), (0, 0)))
    B2 = jnp.pad(W_msg_c2v[D:], ((0, 4), (0, 0)))

    P1 = _matmul(x_vals, W_msg_v2c[:D], zb, 1000)
    Q1 = _matmul(e1, B1, zb, 2048)
    R1 = _matmul(x_cons, W_root_v2c, b_v2c, 1000)
    Q2 = _matmul(e2, B2, zb, 2048)
    R2 = _matmul(x_vals, W_root_c2v, b_c2v, 1000)

    def _prep_src(row):
        p = jnp.pad(row.astype(jnp.int32), (0, E_PAD - E_TOT))
        return p.reshape(NW, CH, K)

    _pad_dst = DUMMY + (jnp.arange(E_PAD - E_TOT, dtype=jnp.int32)
                        % (ACC_ROWS - N))

    def _prep_dst(row):
        p = jnp.concatenate([row.astype(jnp.int32), _pad_dst])
        return p.reshape(NW, CH, K)

    src1 = _prep_src(edge_index_v2c[0])
    dst1 = _prep_dst(edge_index_v2c[1])
    src2 = _prep_src(edge_index_c2v[0])
    dst2 = _prep_dst(edge_index_c2v[1])

    part1, cnt1 = _sc_edge_pass(P1, Q1, src1, dst1)
    x_cons_new = _combine(R1, part1, cnt1)

    P2 = _matmul(x_cons_new, W_msg_c2v[:D], zb, 1000)
    part2, cnt2 = _sc_edge_pass(P2, Q2, src2, dst2)
    x_vals_new = _combine(R2, part2, cnt2)

    return (x_vals_new, x_cons_new)
